# baseline (speedup 1.0000x reference)
import functools

import jax
import jax.numpy as jnp
from jax import lax
from jax.experimental import pallas as pl
from jax.experimental.pallas import tpu as pltpu
from jax.experimental.pallas import tpu_sc as plsc

_B, _L, _D = 4096, 200, 32
_NC, _NS = 2, 16
_NW = _NC * _NS
_NB = _B // _NW
_HALF = _L // 2
_LP = 224


def _make_emb():
    mesh = plsc.VectorSubcoreMesh(core_axis_name="c", subcore_axis_name="s")

    @functools.partial(
        pl.kernel,
        mesh=mesh,
        out_type=jax.ShapeDtypeStruct((_B, _D, _L), jnp.float32),
        compiler_params=pltpu.CompilerParams(
            use_tc_tiling_on_sc=False, needs_layout_passes=False),
        scratch_types=[
            pltpu.VMEM((2 * _NB, _HALF), jnp.int32),
            pltpu.VMEM((_LP, _D), jnp.float32),
            pltpu.VMEM((_D, _LP), jnp.float32),
            pltpu.SemaphoreType.DMA,
        ],
    )
    def emb(x2, table, out, idx_v, rows_v, trans_v, sem_g):
        wid = lax.axis_index("s") * _NC + lax.axis_index("c")
        base = wid * _NB
        pltpu.sync_copy(x2.at[pl.ds(base * 2, 2 * _NB)], idx_v)

        iota = lax.iota(jnp.int32, 16)

        def batch_body(i, carry):
            h0 = pltpu.async_copy(
                table.at[idx_v.at[2 * i]], rows_v.at[pl.ds(0, _HALF)], sem_g)
            h1 = pltpu.async_copy(
                table.at[idx_v.at[2 * i + 1]], rows_v.at[pl.ds(_HALF, _HALF)],
                sem_g)
            h0.wait()
            h1.wait()
            for lb in range(_L // 16 + 1):
                lvec = iota + (16 * lb)
                for d in range(_D):
                    dvec = jnp.full((16,), d, jnp.int32)
                    v = plsc.load_gather(rows_v, [lvec, dvec])
                    trans_v[d, pl.ds(16 * lb, 16)] = v
            pltpu.sync_copy(trans_v.at[:, pl.ds(0, _L)], out.at[base + i])
            return carry

        lax.fori_loop(0, _NB, batch_body, 0)

    return emb


_EMB = _make_emb()


def kernel(x, embedding_table):
    x2 = x.reshape(_B * 2, _HALF).astype(jnp.int32)
    return _EMB(x2, embedding_table)


# speedup vs baseline: 1.0975x; 1.0975x over previous
import functools

import jax
import jax.numpy as jnp
from jax import lax
from jax.experimental import pallas as pl
from jax.experimental.pallas import tpu as pltpu
from jax.experimental.pallas import tpu_sc as plsc

_B, _L, _D = 4096, 200, 32
_NC, _NS = 2, 16
_NW = _NC * _NS
_NB = _B // _NW
_HALF = _L // 2
_LP = 224
_NBUF = 4


def _make_emb():
    mesh = plsc.VectorSubcoreMesh(core_axis_name="c", subcore_axis_name="s")

    @functools.partial(
        pl.kernel,
        mesh=mesh,
        out_type=jax.ShapeDtypeStruct((_B, _D, _L), jnp.float32),
        compiler_params=pltpu.CompilerParams(
            use_tc_tiling_on_sc=False, needs_layout_passes=False),
        scratch_types=[
            pltpu.VMEM((2 * _NB, _HALF), jnp.int32),
            pltpu.VMEM((_NBUF, _LP, _D), jnp.float32),
            pltpu.VMEM((_NBUF, _D, _LP), jnp.float32),
            pltpu.SemaphoreType.DMA((_NBUF,)),
            pltpu.SemaphoreType.DMA((_NBUF,)),
        ],
    )
    def emb(x2, table, out, idx_v, rows_v, trans_v, sem_g, sem_o):
        wid = lax.axis_index("s") * _NC + lax.axis_index("c")
        base = wid * _NB
        pltpu.sync_copy(x2.at[pl.ds(base * 2, 2 * _NB)], idx_v)

        def fire_gather(i, k):
            pltpu.async_copy(
                table.at[idx_v.at[2 * i]],
                rows_v.at[k, pl.ds(0, _HALF)], sem_g.at[k])
            pltpu.async_copy(
                table.at[idx_v.at[2 * i + 1]],
                rows_v.at[k, pl.ds(_HALF, _HALF)], sem_g.at[k])

        for k in range(_NBUF):
            fire_gather(k, k)

        iota = lax.iota(jnp.int32, 16)

        def batch_body(i, carry):
            k = lax.rem(i, _NBUF)
            pltpu.make_async_copy(
                table.at[pl.ds(0, _L)], rows_v.at[k, pl.ds(0, _L)],
                sem_g.at[k]).wait()

            @pl.when(i >= _NBUF)
            def _():
                pltpu.make_async_copy(
                    trans_v.at[k, :, pl.ds(0, _L)], out.at[base],
                    sem_o.at[k]).wait()

            for lb in range(_L // 16 + 1):
                lvec = iota + (16 * lb)
                for d in range(_D):
                    dvec = jnp.full((16,), d, jnp.int32)
                    v = plsc.load_gather(rows_v.at[k], [lvec, dvec])
                    trans_v[k, d, pl.ds(16 * lb, 16)] = v
            pltpu.async_copy(
                trans_v.at[k, :, pl.ds(0, _L)], out.at[base + i], sem_o.at[k])

            @pl.when(i + _NBUF < _NB)
            def _():
                fire_gather(i + _NBUF, k)

            return carry

        lax.fori_loop(0, _NB, batch_body, 0)

        for k in range(_NBUF):
            pltpu.make_async_copy(
                trans_v.at[k, :, pl.ds(0, _L)], out.at[base],
                sem_o.at[k]).wait()

    return emb


_EMB = _make_emb()


def kernel(x, embedding_table):
    x2 = x.reshape(_B * 2, _HALF).astype(jnp.int32)
    return _EMB(x2, embedding_table)


# speedup vs baseline: 1.1206x; 1.0211x over previous
import functools

import jax
import jax.numpy as jnp
from jax import lax
from jax.experimental import pallas as pl
from jax.experimental.pallas import tpu as pltpu
from jax.experimental.pallas import tpu_sc as plsc

_B, _L, _D = 4096, 200, 32
_NC, _NS = 2, 16
_NW = _NC * _NS
_NB = _B // _NW
_HALF = _L // 2
_G = 4
_NG = _NB // _G
_NLB = _L // 16


def _make_emb():
    mesh = plsc.VectorSubcoreMesh(core_axis_name="c", subcore_axis_name="s")

    @functools.partial(
        pl.kernel,
        mesh=mesh,
        out_type=jax.ShapeDtypeStruct((_B, _D, _L), jnp.float32),
        compiler_params=pltpu.CompilerParams(
            use_tc_tiling_on_sc=False, needs_layout_passes=False),
        scratch_types=[
            pltpu.VMEM((2 * _NB, _HALF), jnp.int32),
            pltpu.VMEM((2, _G * _L, _D), jnp.float32),
            pltpu.VMEM((2, _G, _D, _L), jnp.float32),
            pltpu.SemaphoreType.DMA((2,)),
            pltpu.SemaphoreType.DMA((2,)),
        ],
    )
    def emb(x2, table, out, idx_v, rows_v, trans_v, sem_g, sem_o):
        wid = lax.axis_index("s") * _NC + lax.axis_index("c")
        base = wid * _NB
        pltpu.sync_copy(x2.at[pl.ds(base * 2, 2 * _NB)], idx_v)

        def fire_gather(g, k):
            for j in range(2 * _G):
                pltpu.async_copy(
                    table.at[idx_v.at[2 * _G * g + j]],
                    rows_v.at[k, pl.ds(_HALF * j, _HALF)], sem_g.at[k])

        fire_gather(0, 0)

        iota = lax.iota(jnp.int32, 16)
        tail = iota + (16 * _NLB)
        tail_mask = tail < _L

        def group_body(g, carry):
            k = lax.rem(g, 2)

            @pl.when(g + 1 < _NG)
            def _():
                fire_gather(g + 1, 1 - k)

            pltpu.make_async_copy(
                table.at[pl.ds(0, _G * _L)], rows_v.at[k], sem_g.at[k]).wait()

            @pl.when(g >= 2)
            def _():
                pltpu.make_async_copy(
                    trans_v.at[k], out.at[pl.ds(0, _G)], sem_o.at[k]).wait()

            for kb in range(_G):
                for d in range(_D):
                    dvec = jnp.full((16,), d, jnp.int32)
                    for lb in range(_NLB):
                        lvec = iota + (_L * kb + 16 * lb)
                        v = plsc.load_gather(rows_v.at[k], [lvec, dvec])
                        trans_v[k, kb, d, pl.ds(16 * lb, 16)] = v
                    lvec = tail + (_L * kb)
                    v = plsc.load_gather(
                        rows_v.at[k], [lvec, dvec], mask=tail_mask)
                    plsc.store_scatter(
                        trans_v.at[k, kb, d], [tail], v, mask=tail_mask)

            pltpu.async_copy(
                trans_v.at[k], out.at[pl.ds(base + _G * g, _G)], sem_o.at[k])
            return carry

        lax.fori_loop(0, _NG, group_body, 0)

        for k in range(2):
            pltpu.make_async_copy(
                trans_v.at[k], out.at[pl.ds(0, _G)], sem_o.at[k]).wait()

    return emb


_EMB = _make_emb()


def kernel(x, embedding_table):
    x2 = x.reshape(_B * 2, _HALF).astype(jnp.int32)
    return _EMB(x2, embedding_table)


# speedup vs baseline: 1.4587x; 1.3017x over previous
import functools

import jax
import jax.numpy as jnp
from jax import lax
from jax.experimental import pallas as pl
from jax.experimental.pallas import tpu as pltpu
from jax.experimental.pallas import tpu_sc as plsc

_B, _L, _D = 4096, 200, 32
_NC, _NS = 2, 16
_NW = _NC * _NS
_NB = _B // _NW
_HALF = _L // 2
_G = 2
_NG = _NB // _G
_LP = 201


def _make_emb():
    mesh = plsc.VectorSubcoreMesh(core_axis_name="c", subcore_axis_name="s")

    @functools.partial(
        pl.kernel,
        mesh=mesh,
        out_type=jax.ShapeDtypeStruct((_B, _D, _L), jnp.float32),
        compiler_params=pltpu.CompilerParams(
            use_tc_tiling_on_sc=False, needs_layout_passes=False),
        scratch_types=[
            pltpu.VMEM((2 * _NB, _HALF), jnp.int32),
            pltpu.VMEM((2, _G * _L, _D), jnp.float32),
            pltpu.VMEM((2, _G, _D, _LP), jnp.float32),
            pltpu.SemaphoreType.DMA((2,)),
            pltpu.SemaphoreType.DMA((2,)),
        ],
    )
    def emb(x2, table, out, idx_v, rows_v, trans_v, sem_g, sem_o):
        wid = lax.axis_index("s") * _NC + lax.axis_index("c")
        base = wid * _NB
        pltpu.sync_copy(x2.at[pl.ds(base * 2, 2 * _NB)], idx_v)

        def fire_gather(g, k):
            for j in range(2 * _G):
                pltpu.async_copy(
                    table.at[idx_v.at[2 * _G * g + j]],
                    rows_v.at[k, pl.ds(_HALF * j, _HALF)], sem_g.at[k])

        fire_gather(0, 0)

        iota = lax.iota(jnp.int32, 16)
        iota_hi = iota + 16

        def group_body(g, carry):
            k = lax.rem(g, 2)

            @pl.when(g + 1 < _NG)
            def _():
                fire_gather(g + 1, 1 - k)

            pltpu.make_async_copy(
                table.at[pl.ds(0, _G * _L)], rows_v.at[k], sem_g.at[k]).wait()

            @pl.when(g >= 2)
            def _():
                pltpu.make_async_copy(
                    trans_v.at[k, :, :, pl.ds(0, _L)], out.at[pl.ds(0, _G)],
                    sem_o.at[k]).wait()

            for kb in range(_G):
                for l in range(_L):
                    v0 = rows_v[k, _L * kb + l, pl.ds(0, 16)]
                    v1 = rows_v[k, _L * kb + l, pl.ds(16, 16)]
                    lvec = jnp.full((16,), l, jnp.int32)
                    plsc.store_scatter(trans_v.at[k, kb], [iota, lvec], v0)
                    plsc.store_scatter(trans_v.at[k, kb], [iota_hi, lvec], v1)

            pltpu.async_copy(
                trans_v.at[k, :, :, pl.ds(0, _L)],
                out.at[pl.ds(base + _G * g, _G)], sem_o.at[k])
            return carry

        lax.fori_loop(0, _NG, group_body, 0)

        for k in range(2):
            pltpu.make_async_copy(
                trans_v.at[k, :, :, pl.ds(0, _L)], out.at[pl.ds(0, _G)],
                sem_o.at[k]).wait()

    return emb


_EMB = _make_emb()


def kernel(x, embedding_table):
    x2 = x.reshape(_B * 2, _HALF).astype(jnp.int32)
    return _EMB(x2, embedding_table)


# speedup vs baseline: 1.4590x; 1.0002x over previous
import functools

import jax
import jax.numpy as jnp
from jax import lax
from jax.experimental import pallas as pl
from jax.experimental.pallas import tpu as pltpu
from jax.experimental.pallas import tpu_sc as plsc

_B, _L, _D = 4096, 200, 32
_NC, _NS = 2, 16
_NW = _NC * _NS
_NB = _B // _NW
_HALF = _L // 2
_G = 2
_NG = _NB // _G
_LP = 201


def _make_emb():
    mesh = plsc.VectorSubcoreMesh(core_axis_name="c", subcore_axis_name="s")

    @functools.partial(
        pl.kernel,
        mesh=mesh,
        out_type=jax.ShapeDtypeStruct((_B, _D, _L), jnp.float32),
        compiler_params=pltpu.CompilerParams(
            use_tc_tiling_on_sc=False, needs_layout_passes=False,
            disable_bounds_checks=True),
        scratch_types=[
            pltpu.VMEM((2 * _NB, _HALF), jnp.int32),
            pltpu.VMEM((2, _G * _L, _D), jnp.float32),
            pltpu.VMEM((2, _G, _D, _LP), jnp.float32),
            pltpu.SemaphoreType.DMA((2,)),
            pltpu.SemaphoreType.DMA((2,)),
        ],
    )
    def emb(x2, table, out, idx_v, rows_v, trans_v, sem_g, sem_o):
        wid = lax.axis_index("s") * _NC + lax.axis_index("c")
        base = wid * _NB
        pltpu.sync_copy(x2.at[pl.ds(base * 2, 2 * _NB)], idx_v)

        def fire_gather(g, k):
            for j in range(2 * _G):
                pltpu.async_copy(
                    table.at[idx_v.at[2 * _G * g + j]],
                    rows_v.at[k, pl.ds(_HALF * j, _HALF)], sem_g.at[k])

        fire_gather(0, 0)

        iota = lax.iota(jnp.int32, 16)
        iota_hi = iota + 16

        def group_body(g, carry):
            k = lax.rem(g, 2)

            @pl.when(g + 1 < _NG)
            def _():
                fire_gather(g + 1, 1 - k)

            pltpu.make_async_copy(
                table.at[pl.ds(0, _G * _L)], rows_v.at[k], sem_g.at[k]).wait()

            @pl.when(g >= 2)
            def _():
                pltpu.make_async_copy(
                    trans_v.at[k, :, :, pl.ds(0, _L)], out.at[pl.ds(0, _G)],
                    sem_o.at[k]).wait()

            for kb in range(_G):
                for l in range(_L):
                    v0 = rows_v[k, _L * kb + l, pl.ds(0, 16)]
                    v1 = rows_v[k, _L * kb + l, pl.ds(16, 16)]
                    lvec = jnp.full((16,), l, jnp.int32)
                    plsc.store_scatter(trans_v.at[k, kb], [iota, lvec], v0)
                    plsc.store_scatter(trans_v.at[k, kb], [iota_hi, lvec], v1)

            pltpu.async_copy(
                trans_v.at[k, :, :, pl.ds(0, _L)],
                out.at[pl.ds(base + _G * g, _G)], sem_o.at[k])
            return carry

        lax.fori_loop(0, _NG, group_body, 0)

        for k in range(2):
            pltpu.make_async_copy(
                trans_v.at[k, :, :, pl.ds(0, _L)], out.at[pl.ds(0, _G)],
                sem_o.at[k]).wait()

    return emb


_EMB = _make_emb()


def kernel(x, embedding_table):
    x2 = x.reshape(_B * 2, _HALF).astype(jnp.int32)
    return _EMB(x2, embedding_table)


# speedup vs baseline: 2.5105x; 1.7207x over previous
import functools

import jax
import jax.numpy as jnp
from jax import lax
from jax.experimental import pallas as pl
from jax.experimental.pallas import tpu as pltpu
from jax.experimental.pallas import tpu_sc as plsc

_B, _L, _D = 4096, 200, 32
_NC, _NS = 2, 16
_NW = _NC * _NS
_BT = _B // _NW
_LT = _L // 8
_NBUF = 4
_BP = 129


def _make_emb():
    mesh = plsc.VectorSubcoreMesh(core_axis_name="c", subcore_axis_name="s")

    @functools.partial(
        pl.kernel,
        mesh=mesh,
        out_type=jax.ShapeDtypeStruct((_D, _LT, _NW, 8, _BT), jnp.float32),
        compiler_params=pltpu.CompilerParams(
            use_tc_tiling_on_sc=False, needs_layout_passes=False,
            disable_bounds_checks=True),
        scratch_types=[
            pltpu.VMEM((_LT, 8, _BT), jnp.int32),
            pltpu.VMEM((_NBUF, _BT, _D), jnp.float32),
            pltpu.VMEM((_NBUF, _D, _BP), jnp.float32),
            pltpu.SemaphoreType.DMA((_NBUF,)),
            pltpu.SemaphoreType.DMA((_NBUF,)),
        ],
    )
    def emb(x4, table, out5, idx_v, rows_v, trans_v, sem_g, sem_o):
        wid = lax.axis_index("s") * _NC + lax.axis_index("c")
        pltpu.sync_copy(x4.at[:, wid], idx_v)

        def fire_gather(l, k):
            pltpu.async_copy(
                table.at[idx_v.at[lax.div(l, 8), lax.rem(l, 8)]],
                rows_v.at[k], sem_g.at[k])

        for k in range(_NBUF):
            fire_gather(k, k)

        iota = lax.iota(jnp.int32, 16)
        iota_hi = iota + 16

        def l_body(l, carry):
            k = lax.rem(l, _NBUF)

            pltpu.make_async_copy(
                table.at[pl.ds(0, _BT)], rows_v.at[k], sem_g.at[k]).wait()

            @pl.when(l >= _NBUF)
            def _():
                pltpu.make_async_copy(
                    trans_v.at[k, :, pl.ds(0, _BT)], out5.at[:, 0, 0, 0],
                    sem_o.at[k]).wait()

            for b in range(_BT):
                v0 = rows_v[k, b, pl.ds(0, 16)]
                v1 = rows_v[k, b, pl.ds(16, 16)]
                bvec = jnp.full((16,), b, jnp.int32)
                plsc.store_scatter(trans_v.at[k], [iota, bvec], v0)
                plsc.store_scatter(trans_v.at[k], [iota_hi, bvec], v1)

            @pl.when(l + _NBUF < _L)
            def _():
                fire_gather(l + _NBUF, k)

            pltpu.async_copy(
                trans_v.at[k, :, pl.ds(0, _BT)],
                out5.at[:, lax.div(l, 8), wid, lax.rem(l, 8)], sem_o.at[k])
            return carry

        lax.fori_loop(0, _L, l_body, 0)

        for k in range(_NBUF):
            pltpu.make_async_copy(
                trans_v.at[k, :, pl.ds(0, _BT)], out5.at[:, 0, 0, 0],
                sem_o.at[k]).wait()

    return emb


_EMB = _make_emb()


def kernel(x, embedding_table):
    x4 = x.astype(jnp.int32).reshape(_NW, _BT, _LT, 8).transpose((2, 0, 3, 1))
    out5 = _EMB(x4, embedding_table)
    return out5.transpose((2, 4, 0, 1, 3)).reshape(_B, _D, _L)


# speedup vs baseline: 2.5108x; 1.0001x over previous
import functools

import jax
import jax.numpy as jnp
from jax import lax
from jax.experimental import pallas as pl
from jax.experimental.pallas import tpu as pltpu
from jax.experimental.pallas import tpu_sc as plsc

_B, _L, _D = 4096, 200, 32
_NC, _NS = 2, 16
_NW = _NC * _NS
_BT = _B // _NW
_LT = _L // 8
_NBUF = 8
_BP = 129


def _make_emb():
    mesh = plsc.VectorSubcoreMesh(core_axis_name="c", subcore_axis_name="s")

    @functools.partial(
        pl.kernel,
        mesh=mesh,
        out_type=jax.ShapeDtypeStruct((_D, _LT, _NW, 8, _BT), jnp.float32),
        compiler_params=pltpu.CompilerParams(
            use_tc_tiling_on_sc=False, needs_layout_passes=False,
            disable_bounds_checks=True),
        scratch_types=[
            pltpu.VMEM((_LT, 8, _BT), jnp.int32),
            pltpu.VMEM((_NBUF, _BT, _D), jnp.float32),
            pltpu.VMEM((_NBUF, _D, _BP), jnp.float32),
            pltpu.SemaphoreType.DMA((_NBUF,)),
            pltpu.SemaphoreType.DMA((_NBUF,)),
        ],
    )
    def emb(x4, table, out5, idx_v, rows_v, trans_v, sem_g, sem_o):
        wid = lax.axis_index("s") * _NC + lax.axis_index("c")
        pltpu.sync_copy(x4.at[:, wid], idx_v)

        def fire_gather(l, k):
            pltpu.async_copy(
                table.at[idx_v.at[lax.div(l, 8), lax.rem(l, 8)]],
                rows_v.at[k], sem_g.at[k])

        for k in range(_NBUF):
            fire_gather(k, k)

        iota = lax.iota(jnp.int32, 16)
        iota_hi = iota + 16

        def l_body(l, carry):
            k = lax.rem(l, _NBUF)

            pltpu.make_async_copy(
                table.at[pl.ds(0, _BT)], rows_v.at[k], sem_g.at[k]).wait()

            @pl.when(l >= _NBUF)
            def _():
                pltpu.make_async_copy(
                    trans_v.at[k, :, pl.ds(0, _BT)], out5.at[:, 0, 0, 0],
                    sem_o.at[k]).wait()

            for b in range(_BT):
                v0 = rows_v[k, b, pl.ds(0, 16)]
                v1 = rows_v[k, b, pl.ds(16, 16)]
                bvec = jnp.full((16,), b, jnp.int32)
                plsc.store_scatter(trans_v.at[k], [iota, bvec], v0)
                plsc.store_scatter(trans_v.at[k], [iota_hi, bvec], v1)

            @pl.when(l + _NBUF < _L)
            def _():
                fire_gather(l + _NBUF, k)

            pltpu.async_copy(
                trans_v.at[k, :, pl.ds(0, _BT)],
                out5.at[:, lax.div(l, 8), wid, lax.rem(l, 8)], sem_o.at[k])
            return carry

        lax.fori_loop(0, _L, l_body, 0)

        for k in range(_NBUF):
            pltpu.make_async_copy(
                trans_v.at[k, :, pl.ds(0, _BT)], out5.at[:, 0, 0, 0],
                sem_o.at[k]).wait()

    return emb


_EMB = _make_emb()


def kernel(x, embedding_table):
    x4 = x.astype(jnp.int32).reshape(_NW, _BT, _LT, 8).transpose((2, 0, 3, 1))
    out5 = _EMB(x4, embedding_table)
    return out5.transpose((2, 4, 0, 1, 3)).reshape(_B, _D, _L)


# speedup vs baseline: 2.5134x; 1.0011x over previous
import functools

import jax
import jax.numpy as jnp
from jax import lax
from jax.experimental import pallas as pl
from jax.experimental.pallas import tpu as pltpu
from jax.experimental.pallas import tpu_sc as plsc

_B, _L, _D = 4096, 200, 32
_NC, _NS = 2, 16
_NW = _NC * _NS
_BT = _B // _NW
_LT = _L // 8
_NBUF = 4
_BP = 129


def _make_emb():
    mesh = plsc.VectorSubcoreMesh(core_axis_name="c", subcore_axis_name="s")

    @functools.partial(
        pl.kernel,
        mesh=mesh,
        out_type=jax.ShapeDtypeStruct((_D, _LT, _NW, 8, _BT), jnp.float32),
        compiler_params=pltpu.CompilerParams(
            use_tc_tiling_on_sc=False, needs_layout_passes=False,
            disable_bounds_checks=True),
        scratch_types=[
            pltpu.VMEM((_LT, 8, _BT), jnp.int32),
            pltpu.VMEM((_NBUF, _BT, _D), jnp.float32),
            pltpu.VMEM((_NBUF, _D, _BP), jnp.float32),
            pltpu.SemaphoreType.DMA((_NBUF,)),
            pltpu.SemaphoreType.DMA((_NBUF,)),
        ],
    )
    def emb(x4, table, out5, idx_v, rows_v, trans_v, sem_g, sem_o):
        wid = lax.axis_index("s") * _NC + lax.axis_index("c")
        pltpu.sync_copy(x4.at[:, wid], idx_v)

        def fire_gather(l, k):
            pltpu.async_copy(
                table.at[idx_v.at[lax.div(l, 8), lax.rem(l, 8)]],
                rows_v.at[k], sem_g.at[k])

        for k in range(_NBUF):
            fire_gather(k, k)

        iota = lax.iota(jnp.int32, 16)
        iota_hi = iota + 16

        def l_body(l, carry):
            k = lax.rem(l, _NBUF)

            pltpu.make_async_copy(
                table.at[pl.ds(0, _BT)], rows_v.at[k], sem_g.at[k]).wait()

            @pl.when(l >= _NBUF)
            def _():
                pltpu.make_async_copy(
                    trans_v.at[k, :, pl.ds(0, _BT)], out5.at[:, 0, 0, 0],
                    sem_o.at[k]).wait()

            for b in range(_BT):
                v0 = rows_v[k, b, pl.ds(0, 16)]
                v1 = rows_v[k, b, pl.ds(16, 16)]
                bvec = jnp.full((16,), b, jnp.int32)
                plsc.store_scatter(trans_v.at[k], [iota, bvec], v0)
                plsc.store_scatter(trans_v.at[k], [iota_hi, bvec], v1)

            @pl.when(l + _NBUF < _L)
            def _():
                fire_gather(l + _NBUF, k)

            pltpu.async_copy(
                trans_v.at[k, :, pl.ds(0, _BT)],
                out5.at[:, lax.div(l, 8), wid, lax.rem(l, 8)], sem_o.at[k])
            return carry

        lax.fori_loop(0, _L, l_body, 0)

        for k in range(_NBUF):
            pltpu.make_async_copy(
                trans_v.at[k, :, pl.ds(0, _BT)], out5.at[:, 0, 0, 0],
                sem_o.at[k]).wait()

    return emb


_EMB = _make_emb()


def kernel(x, embedding_table):
    x4 = x.astype(jnp.int32).reshape(_NW, _BT, _LT, 8).transpose((2, 0, 3, 1))
    out5 = _EMB(x4, embedding_table)
    return out5.transpose((2, 4, 0, 1, 3)).reshape(_B, _D, _L)
